# baseline (device time: 15047 ns/iter reference)
import jax
import jax.numpy as jnp
from jax import lax
from jax.experimental import pallas as pl
from jax.experimental.pallas import tpu as pltpu

N_DEV = 4
ISSUE_ORDER = (2, 1, 3)
WAIT_ORDER = (1, 3, 2)


def kernel(x):
    m, n = x.shape
    mc = m // N_DEV

    def body(
        x_ref,
        out_ref,
        xb_ref,
        ag_buf,
        rs_recv,
        ag_recv,
        rs_send_sems,
        rs_recv_sems,
        ag_send_sems,
        ag_recv_sems,
    ):
        my = lax.axis_index("i")

        barrier_sem = pltpu.get_barrier_semaphore()
        for o in (1, 2, 3):
            peer = (my + o) % N_DEV
            pl.semaphore_signal(
                barrier_sem, inc=1,
                device_id=(peer,), device_id_type=pl.DeviceIdType.MESH,
            )
        pl.semaphore_wait(barrier_sem, 3)

        xb_ref[...] = x_ref[...].astype(jnp.bfloat16)


        rs_rdmas = {}
        for o in ISSUE_ORDER:
            peer = (my + o) % N_DEV
            slot = 3 - o
            r = pltpu.make_async_remote_copy(
                src_ref=xb_ref.at[pl.ds(peer * mc, mc), :],
                dst_ref=rs_recv.at[slot],
                send_sem=rs_send_sems.at[slot],
                recv_sem=rs_recv_sems.at[slot],
                device_id=(peer,),
                device_id_type=pl.DeviceIdType.MESH,
            )
            r.start()
            rs_rdmas[o] = r

        acc = x_ref[pl.ds(my * mc, mc), :]
        for o in WAIT_ORDER:
            slot = 3 - o
            rs_rdmas[o].wait_recv()
            acc = acc + rs_recv[slot].astype(jnp.float32)
        ag_buf[...] = acc.astype(jnp.bfloat16)

        ag_rdmas = {}
        for o in ISSUE_ORDER:
            peer = (my + o) % N_DEV
            slot = 3 - o
            r = pltpu.make_async_remote_copy(
                src_ref=ag_buf,
                dst_ref=ag_recv.at[slot],
                send_sem=ag_send_sems.at[slot],
                recv_sem=ag_recv_sems.at[slot],
                device_id=(peer,),
                device_id_type=pl.DeviceIdType.MESH,
            )
            r.start()
            ag_rdmas[o] = r

        out_ref[pl.ds(my * mc, mc), :] = acc

        for o in WAIT_ORDER:
            slot = 3 - o
            ag_rdmas[o].wait_recv()
            origin = (my + slot + 1) % N_DEV
            out_ref[pl.ds(origin * mc, mc), :] = ag_recv[slot].astype(jnp.float32)

        for o in (1, 2, 3):
            rs_rdmas[o].wait_send()
            ag_rdmas[o].wait_send()

    return pl.pallas_call(
        body,
        out_shape=jax.ShapeDtypeStruct((m, n), jnp.float32),
        in_specs=[pl.BlockSpec(memory_space=pltpu.VMEM)],
        out_specs=pl.BlockSpec(memory_space=pltpu.VMEM),
        scratch_shapes=[
            pltpu.VMEM((m, n), jnp.bfloat16),
            pltpu.VMEM((mc, n), jnp.bfloat16),
            pltpu.VMEM((N_DEV - 1, mc, n), jnp.bfloat16),
            pltpu.VMEM((N_DEV - 1, mc, n), jnp.bfloat16),
            pltpu.SemaphoreType.DMA((N_DEV - 1,)),
            pltpu.SemaphoreType.DMA((N_DEV - 1,)),
            pltpu.SemaphoreType.DMA((N_DEV - 1,)),
            pltpu.SemaphoreType.DMA((N_DEV - 1,)),
        ],
        compiler_params=pltpu.CompilerParams(collective_id=0),
    )(x)


# device time: 13715 ns/iter; 1.0971x vs baseline; 1.0971x over previous
import jax
import jax.numpy as jnp
from jax import lax
from jax.experimental import pallas as pl
from jax.experimental.pallas import tpu as pltpu

N_DEV = 4
N_HALF = 2
ISSUE_ORDER = (2, 1, 3)
WAIT_ORDER = (1, 3, 2)


def kernel(x):
    m, n = x.shape
    mc = m // N_DEV
    hc = mc // N_HALF

    def body(
        x_ref,
        out_ref,
        xb_ref,
        ag_buf,
        rs_recv,
        ag_recv,
        rs_send_sems,
        rs_recv_sems,
        ag_send_sems,
        ag_recv_sems,
    ):
        my = lax.axis_index("i")

        barrier_sem = pltpu.get_barrier_semaphore()
        for o in (1, 2, 3):
            peer = (my + o) % N_DEV
            pl.semaphore_signal(
                barrier_sem, inc=1,
                device_id=(peer,), device_id_type=pl.DeviceIdType.MESH,
            )
        pl.semaphore_wait(barrier_sem, 3)

        xb_ref[...] = x_ref[...].astype(jnp.bfloat16)


        rs_rdmas = {}
        for h in range(N_HALF):
            for o in ISSUE_ORDER:
                peer = (my + o) % N_DEV
                slot = 3 - o
                r = pltpu.make_async_remote_copy(
                    src_ref=xb_ref.at[pl.ds(peer * mc + h * hc, hc), :],
                    dst_ref=rs_recv.at[h, slot],
                    send_sem=rs_send_sems.at[h, slot],
                    recv_sem=rs_recv_sems.at[h, slot],
                    device_id=(peer,),
                    device_id_type=pl.DeviceIdType.MESH,
                )
                r.start()
                rs_rdmas[h, o] = r

        ag_rdmas = {}
        for h in range(N_HALF):
            acc = x_ref[pl.ds(my * mc + h * hc, hc), :]
            for o in WAIT_ORDER:
                slot = 3 - o
                rs_rdmas[h, o].wait_recv()
                acc = acc + rs_recv[h, slot].astype(jnp.float32)
            ag_buf[h] = acc.astype(jnp.bfloat16)
            for o in ISSUE_ORDER:
                peer = (my + o) % N_DEV
                slot = 3 - o
                r = pltpu.make_async_remote_copy(
                    src_ref=ag_buf.at[h],
                    dst_ref=ag_recv.at[h, slot],
                    send_sem=ag_send_sems.at[h, slot],
                    recv_sem=ag_recv_sems.at[h, slot],
                    device_id=(peer,),
                    device_id_type=pl.DeviceIdType.MESH,
                )
                r.start()
                ag_rdmas[h, o] = r
            out_ref[pl.ds(my * mc + h * hc, hc), :] = acc

        for h in range(N_HALF):
            for o in WAIT_ORDER:
                slot = 3 - o
                ag_rdmas[h, o].wait_recv()
                origin = (my + slot + 1) % N_DEV
                out_ref[pl.ds(origin * mc + h * hc, hc), :] = (
                    ag_recv[h, slot].astype(jnp.float32)
                )

        for h in range(N_HALF):
            for o in (1, 2, 3):
                rs_rdmas[h, o].wait_send()
                ag_rdmas[h, o].wait_send()

    return pl.pallas_call(
        body,
        out_shape=jax.ShapeDtypeStruct((m, n), jnp.float32),
        in_specs=[pl.BlockSpec(memory_space=pltpu.VMEM)],
        out_specs=pl.BlockSpec(memory_space=pltpu.VMEM),
        scratch_shapes=[
            pltpu.VMEM((m, n), jnp.bfloat16),
            pltpu.VMEM((N_HALF, hc, n), jnp.bfloat16),
            pltpu.VMEM((N_HALF, N_DEV - 1, hc, n), jnp.bfloat16),
            pltpu.VMEM((N_HALF, N_DEV - 1, hc, n), jnp.bfloat16),
            pltpu.SemaphoreType.DMA((N_HALF, N_DEV - 1)),
            pltpu.SemaphoreType.DMA((N_HALF, N_DEV - 1)),
            pltpu.SemaphoreType.DMA((N_HALF, N_DEV - 1)),
            pltpu.SemaphoreType.DMA((N_HALF, N_DEV - 1)),
        ],
        compiler_params=pltpu.CompilerParams(collective_id=0),
    )(x)


# device time: 13518 ns/iter; 1.1131x vs baseline; 1.0146x over previous
import jax
import jax.numpy as jnp
from jax import lax
from jax.experimental import pallas as pl
from jax.experimental.pallas import tpu as pltpu

N_DEV = 4
N_HALF = 4
ISSUE_ORDER = (2, 1, 3)
WAIT_ORDER = (1, 3, 2)


def kernel(x):
    m, n = x.shape
    mc = m // N_DEV
    hc = mc // N_HALF

    def body(
        x_ref,
        out_ref,
        xb_ref,
        ag_buf,
        rs_recv,
        ag_recv,
        rs_send_sems,
        rs_recv_sems,
        ag_send_sems,
        ag_recv_sems,
    ):
        my = lax.axis_index("i")

        barrier_sem = pltpu.get_barrier_semaphore()
        for o in (1, 2, 3):
            peer = (my + o) % N_DEV
            pl.semaphore_signal(
                barrier_sem, inc=1,
                device_id=(peer,), device_id_type=pl.DeviceIdType.MESH,
            )
        pl.semaphore_wait(barrier_sem, 3)

        xb_ref[...] = x_ref[...].astype(jnp.bfloat16)


        rs_rdmas = {}
        for h in range(N_HALF):
            for o in ISSUE_ORDER:
                peer = (my + o) % N_DEV
                slot = 3 - o
                r = pltpu.make_async_remote_copy(
                    src_ref=xb_ref.at[pl.ds(peer * mc + h * hc, hc), :],
                    dst_ref=rs_recv.at[h, slot],
                    send_sem=rs_send_sems.at[h, slot],
                    recv_sem=rs_recv_sems.at[h, slot],
                    device_id=(peer,),
                    device_id_type=pl.DeviceIdType.MESH,
                )
                r.start()
                rs_rdmas[h, o] = r

        ag_rdmas = {}
        for h in range(N_HALF):
            acc = x_ref[pl.ds(my * mc + h * hc, hc), :]
            for o in WAIT_ORDER:
                slot = 3 - o
                rs_rdmas[h, o].wait_recv()
                acc = acc + rs_recv[h, slot].astype(jnp.float32)
            ag_buf[h] = acc.astype(jnp.bfloat16)
            for o in ISSUE_ORDER:
                peer = (my + o) % N_DEV
                slot = 3 - o
                r = pltpu.make_async_remote_copy(
                    src_ref=ag_buf.at[h],
                    dst_ref=ag_recv.at[h, slot],
                    send_sem=ag_send_sems.at[h, slot],
                    recv_sem=ag_recv_sems.at[h, slot],
                    device_id=(peer,),
                    device_id_type=pl.DeviceIdType.MESH,
                )
                r.start()
                ag_rdmas[h, o] = r
            out_ref[pl.ds(my * mc + h * hc, hc), :] = acc

        for h in range(N_HALF):
            for o in WAIT_ORDER:
                slot = 3 - o
                ag_rdmas[h, o].wait_recv()
                origin = (my + slot + 1) % N_DEV
                out_ref[pl.ds(origin * mc + h * hc, hc), :] = (
                    ag_recv[h, slot].astype(jnp.float32)
                )

        for h in range(N_HALF):
            for o in (1, 2, 3):
                rs_rdmas[h, o].wait_send()
                ag_rdmas[h, o].wait_send()

    return pl.pallas_call(
        body,
        out_shape=jax.ShapeDtypeStruct((m, n), jnp.float32),
        in_specs=[pl.BlockSpec(memory_space=pltpu.VMEM)],
        out_specs=pl.BlockSpec(memory_space=pltpu.VMEM),
        scratch_shapes=[
            pltpu.VMEM((m, n), jnp.bfloat16),
            pltpu.VMEM((N_HALF, hc, n), jnp.bfloat16),
            pltpu.VMEM((N_HALF, N_DEV - 1, hc, n), jnp.bfloat16),
            pltpu.VMEM((N_HALF, N_DEV - 1, hc, n), jnp.bfloat16),
            pltpu.SemaphoreType.DMA((N_HALF, N_DEV - 1)),
            pltpu.SemaphoreType.DMA((N_HALF, N_DEV - 1)),
            pltpu.SemaphoreType.DMA((N_HALF, N_DEV - 1)),
            pltpu.SemaphoreType.DMA((N_HALF, N_DEV - 1)),
        ],
        compiler_params=pltpu.CompilerParams(collective_id=0),
    )(x)
